# baseline (device time: 423259 ns/iter reference)
import jax
import jax.numpy as jnp
from jax import lax
from jax.experimental import pallas as pl
from jax.experimental.pallas import tpu as pltpu

N_DEV = 32
PLANE = 8
N_PLANES = N_DEV // PLANE
SQ_L = 256
SKV_L = 256
HQ = 4
DH = 64
BLK = 64


def kernel(x, Wq, K_ext, V_ext, Wo):
    B = x.shape[0]

    kvt = jnp.stack([K_ext, V_ext]).astype(jnp.bfloat16).transpose(0, 1, 3, 2, 4)

    def body(x_ref, wq_ref, kvt_ref, wo_ref, out_ref, kvg_ref,
             p1s_sems, p1r_sems, us_sems, ur_sems, ds_sems, dr_sems):
        my = lax.axis_index("i")
        my_p = my % PLANE
        pbase = (my // PLANE) * PLANE
        my_z = my // PLANE
        zup = (my + PLANE) % N_DEV
        zdn = (my + N_DEV - PLANE) % N_DEV

        barrier_sem = pltpu.get_barrier_semaphore()
        mates = [pbase + (my_p + j) % PLANE for j in range(1, PLANE)]
        for nbr in mates + [zup, zdn]:
            pl.semaphore_signal(
                barrier_sem, inc=1,
                device_id=(nbr,), device_id_type=pl.DeviceIdType.MESH,
            )
        pl.semaphore_wait(barrier_sem, PLANE + 1)

        kvg_ref[my] = kvt_ref[...]

        p1 = []
        for j in range(1, PLANE):
            d = pltpu.make_async_remote_copy(
                src_ref=kvg_ref.at[my],
                dst_ref=kvg_ref.at[my],
                send_sem=p1s_sems.at[j - 1],
                recv_sem=p1r_sems.at[j - 1],
                device_id=(mates[j - 1],),
                device_id_type=pl.DeviceIdType.MESH,
            )
            d.start()
            p1.append(d)
        for d in p1:
            d.wait_send()
        for d in p1:
            d.wait_recv()

        zt = N_PLANES - 1
        U, D = [], []
        for r in range(N_PLANES - 1):
            s = r % 2
            bu = jnp.maximum(my_z - r, 0)
            U.append(pltpu.make_async_remote_copy(
                src_ref=kvg_ref.at[pl.ds(bu * PLANE, PLANE)],
                dst_ref=kvg_ref.at[pl.ds(bu * PLANE, PLANE)],
                send_sem=us_sems.at[s], recv_sem=ur_sems.at[s],
                device_id=(zup,), device_id_type=pl.DeviceIdType.MESH,
            ))
            bd = jnp.minimum(my_z + r, zt)
            D.append(pltpu.make_async_remote_copy(
                src_ref=kvg_ref.at[pl.ds(bd * PLANE, PLANE)],
                dst_ref=kvg_ref.at[pl.ds(bd * PLANE, PLANE)],
                send_sem=ds_sems.at[s], recv_sem=dr_sems.at[s],
                device_id=(zdn,), device_id_type=pl.DeviceIdType.MESH,
            ))
        sU = [(my_z < zt) & (my_z - r >= 0) for r in range(3)]
        sD = [(my_z > 0) & (my_z + r <= zt) for r in range(3)]
        rU = [my_z >= r + 1 for r in range(3)]
        rD = [my_z + r + 1 <= zt for r in range(3)]

        def _when(cond, *ops):
            @pl.when(cond)
            def _():
                for op in ops:
                    op()

        _when(sU[0], U[0].start)
        _when(sD[0], D[0].start)
        _when(rU[0], U[0].wait_recv)
        _when(sU[1], U[1].start)
        _when(rD[0], D[0].wait_recv)
        _when(sD[1], D[1].start)
        _when(rU[1], U[1].wait_recv)
        _when(sU[2], U[0].wait_send, U[2].start)
        _when(rD[1], D[1].wait_recv)
        _when(sD[2], D[0].wait_send, D[2].start)
        _when(rU[2], U[2].wait_recv)
        _when(rD[2], D[2].wait_recv)
        _when(sU[0] & jnp.logical_not(sU[2]), U[0].wait_send)
        _when(sD[0] & jnp.logical_not(sD[2]), D[0].wait_send)
        _when(sU[1], U[1].wait_send)
        _when(sD[1], D[1].wait_send)
        _when(sU[2], U[2].wait_send)
        _when(sD[2], D[2].wait_send)

        base = my * SQ_L
        wq_b = wq_ref[...].astype(jnp.bfloat16)
        wo_b = wo_ref[...].astype(jnp.bfloat16)
        qb = (base + lax.broadcasted_iota(jnp.int32, (SQ_L, SKV_L), 0)) // BLK
        cb = lax.broadcasted_iota(jnp.int32, (SQ_L, SKV_L), 1) // BLK

        qs = []
        for b in range(B):
            q_all = jnp.dot(
                x_ref[b].astype(jnp.bfloat16), wq_b,
                preferred_element_type=jnp.float32,
            )
            for h in range(HQ):
                qs.append(q_all[:, h * DH:(h + 1) * DH].astype(jnp.bfloat16))

        def chunk_step(o, carry):
            accs, ls = carry
            kb = cb + (o * (SKV_L // BLK))
            mask = (qb == kb) | (kb == 0) | ((qb + kb) % 3 == 0)
            new_accs = []
            new_ls = []
            for b in range(B):
                for h in range(HQ):
                    i = b * HQ + h
                    k = kvg_ref[o, 0, b, h]
                    v = kvg_ref[o, 1, b, h]
                    sc = lax.dot_general(
                        qs[i], k, (((1,), (1,)), ((), ())),
                        preferred_element_type=jnp.float32,
                    ) * 0.125
                    w = jnp.where(mask, jnp.exp(sc), 0.0)
                    new_accs.append(accs[i] + lax.dot_general(
                        w.astype(jnp.bfloat16), v, (((1,), (0,)), ((), ())),
                        preferred_element_type=jnp.float32,
                    ))
                    new_ls.append(ls[i] + jnp.sum(w, axis=1, keepdims=True))
            return tuple(new_accs), tuple(new_ls)

        zero_accs = tuple(jnp.zeros((SQ_L, DH), jnp.float32) for _ in range(B * HQ))
        zero_ls = tuple(jnp.zeros((SQ_L, 1), jnp.float32) for _ in range(B * HQ))
        accs, ls = lax.fori_loop(0, N_DEV, chunk_step, (zero_accs, zero_ls))

        for b in range(B):
            ctx = jnp.concatenate(
                [accs[b * HQ + h] / ls[b * HQ + h] for h in range(HQ)], axis=1
            ).astype(jnp.bfloat16)
            out_ref[b] = jnp.dot(ctx, wo_b, preferred_element_type=jnp.float32)

    return pl.pallas_call(
        body,
        out_shape=jax.ShapeDtypeStruct((B, SQ_L, HQ * DH * 2), jnp.float32),
        in_specs=[
            pl.BlockSpec(memory_space=pltpu.VMEM),
            pl.BlockSpec(memory_space=pltpu.VMEM),
            pl.BlockSpec(memory_space=pltpu.VMEM),
            pl.BlockSpec(memory_space=pltpu.VMEM),
        ],
        out_specs=pl.BlockSpec(memory_space=pltpu.VMEM),
        scratch_shapes=[
            pltpu.VMEM((N_DEV, 2, B, HQ, SKV_L, DH), jnp.bfloat16),
            pltpu.SemaphoreType.DMA((PLANE - 1,)),
            pltpu.SemaphoreType.DMA((PLANE - 1,)),
            pltpu.SemaphoreType.DMA((2,)),
            pltpu.SemaphoreType.DMA((2,)),
            pltpu.SemaphoreType.DMA((2,)),
            pltpu.SemaphoreType.DMA((2,)),
        ],
        compiler_params=pltpu.CompilerParams(
            collective_id=0, vmem_limit_bytes=100 * 1024 * 1024
        ),
    )(x, Wq, kvt, Wo)


# device time: 358177 ns/iter; 1.1817x vs baseline; 1.1817x over previous
import jax
import jax.numpy as jnp
from jax import lax
from jax.experimental import pallas as pl
from jax.experimental.pallas import tpu as pltpu

N_DEV = 32
PLANE = 8
N_PLANES = N_DEV // PLANE
SQ_L = 256
SKV_L = 256
HQ = 4
DH = 64
BLK = 64


def kernel(x, Wq, K_ext, V_ext, Wo):
    B = x.shape[0]

    kvt = jnp.stack([K_ext, V_ext]).astype(jnp.bfloat16).transpose(0, 1, 3, 2, 4)

    def body(x_ref, wq_ref, kvt_ref, wo_ref, out_ref, kvg_ref,
             p1s_sems, p1r_sems, u0s_sems, u0r_sems, d0s_sems, d0r_sems,
             us_sems, ur_sems, ds_sems, dr_sems):
        my = lax.axis_index("i")
        my_p = my % PLANE
        pbase = (my // PLANE) * PLANE
        my_z = my // PLANE
        zup = (my + PLANE) % N_DEV
        zdn = (my + N_DEV - PLANE) % N_DEV

        barrier_sem = pltpu.get_barrier_semaphore()
        mates = [pbase + (my_p + j) % PLANE for j in range(1, PLANE)]
        for nbr in mates + [zup, zdn]:
            pl.semaphore_signal(
                barrier_sem, inc=1,
                device_id=(nbr,), device_id_type=pl.DeviceIdType.MESH,
            )
        pl.semaphore_wait(barrier_sem, PLANE + 1)

        kvg_ref[my] = kvt_ref[...]

        zt = N_PLANES - 1
        up_ok = my_z < zt
        dn_ok = my_z > 0

        def _when(cond, *ops):
            @pl.when(cond)
            def _():
                for op in ops:
                    op()

        def _chunk_z(slot, origin):
            u = pltpu.make_async_remote_copy(
                src_ref=kvg_ref.at[origin], dst_ref=kvg_ref.at[origin],
                send_sem=u0s_sems.at[slot], recv_sem=u0r_sems.at[slot],
                device_id=(zup,), device_id_type=pl.DeviceIdType.MESH,
            )
            d = pltpu.make_async_remote_copy(
                src_ref=kvg_ref.at[origin], dst_ref=kvg_ref.at[origin],
                send_sem=d0s_sems.at[slot], recv_sem=d0r_sems.at[slot],
                device_id=(zdn,), device_id_type=pl.DeviceIdType.MESH,
            )
            return u, d

        zc = [_chunk_z(0, my)]
        _when(up_ok, zc[0][0].start)
        _when(dn_ok, zc[0][1].start)

        p1 = []
        for j in range(1, PLANE):
            d = pltpu.make_async_remote_copy(
                src_ref=kvg_ref.at[my],
                dst_ref=kvg_ref.at[my],
                send_sem=p1s_sems.at[j - 1],
                recv_sem=p1r_sems.at[j - 1],
                device_id=(mates[j - 1],),
                device_id_type=pl.DeviceIdType.MESH,
            )
            d.start()
            p1.append(d)
        for j in range(1, PLANE):
            p1[j - 1].wait_recv()
            mj = pbase + (my_p + PLANE - j) % PLANE
            zc.append(_chunk_z(j, mj))
            _when(up_ok, zc[j][0].start)
            _when(dn_ok, zc[j][1].start)
        for d in p1:
            d.wait_send()

        U, D = [None], [None]
        for r in (1, 2):
            s = r % 2
            bu = jnp.maximum(my_z - r, 0)
            U.append(pltpu.make_async_remote_copy(
                src_ref=kvg_ref.at[pl.ds(bu * PLANE, PLANE)],
                dst_ref=kvg_ref.at[pl.ds(bu * PLANE, PLANE)],
                send_sem=us_sems.at[s], recv_sem=ur_sems.at[s],
                device_id=(zup,), device_id_type=pl.DeviceIdType.MESH,
            ))
            bd = jnp.minimum(my_z + r, zt)
            D.append(pltpu.make_async_remote_copy(
                src_ref=kvg_ref.at[pl.ds(bd * PLANE, PLANE)],
                dst_ref=kvg_ref.at[pl.ds(bd * PLANE, PLANE)],
                send_sem=ds_sems.at[s], recv_sem=dr_sems.at[s],
                device_id=(zdn,), device_id_type=pl.DeviceIdType.MESH,
            ))
        sU = [up_ok, up_ok & (my_z >= 1), up_ok & (my_z >= 2)]
        sD = [dn_ok, dn_ok & (my_z + 1 <= zt), dn_ok & (my_z + 2 <= zt)]
        rU = [my_z >= r + 1 for r in range(3)]
        rD = [my_z + r + 1 <= zt for r in range(3)]

        for j in range(PLANE):
            _when(rU[0], zc[j][0].wait_recv)
        _when(sU[1], U[1].start)
        for j in range(PLANE):
            _when(rD[0], zc[j][1].wait_recv)
        _when(sD[1], D[1].start)
        _when(rU[1], U[1].wait_recv)
        _when(sU[2], U[2].start)
        _when(rD[1], D[1].wait_recv)
        _when(sD[2], D[2].start)
        _when(rU[2], U[2].wait_recv)
        _when(rD[2], D[2].wait_recv)
        for j in range(PLANE):
            _when(up_ok, zc[j][0].wait_send)
            _when(dn_ok, zc[j][1].wait_send)
        _when(sU[1], U[1].wait_send)
        _when(sD[1], D[1].wait_send)
        _when(sU[2], U[2].wait_send)
        _when(sD[2], D[2].wait_send)

        base = my * SQ_L
        wq_b = wq_ref[...].astype(jnp.bfloat16)
        wo_b = wo_ref[...].astype(jnp.bfloat16)
        qb = (base + lax.broadcasted_iota(jnp.int32, (SQ_L, SKV_L), 0)) // BLK
        cb = lax.broadcasted_iota(jnp.int32, (SQ_L, SKV_L), 1) // BLK

        qs = []
        for b in range(B):
            q_all = jnp.dot(
                x_ref[b].astype(jnp.bfloat16), wq_b,
                preferred_element_type=jnp.float32,
            )
            for h in range(HQ):
                qs.append(q_all[:, h * DH:(h + 1) * DH].astype(jnp.bfloat16))

        def chunk_step(o, carry):
            accs, ls = carry
            kb = cb + (o * (SKV_L // BLK))
            mask = (qb == kb) | (kb == 0) | ((qb + kb) % 3 == 0)
            new_accs = []
            new_ls = []
            for b in range(B):
                for h in range(HQ):
                    i = b * HQ + h
                    k = kvg_ref[o, 0, b, h]
                    v = kvg_ref[o, 1, b, h]
                    sc = lax.dot_general(
                        qs[i], k, (((1,), (1,)), ((), ())),
                        preferred_element_type=jnp.float32,
                    ) * 0.125
                    w = jnp.where(mask, jnp.exp(sc), 0.0)
                    new_accs.append(accs[i] + lax.dot_general(
                        w.astype(jnp.bfloat16), v, (((1,), (0,)), ((), ())),
                        preferred_element_type=jnp.float32,
                    ))
                    new_ls.append(ls[i] + jnp.sum(w, axis=1, keepdims=True))
            return tuple(new_accs), tuple(new_ls)

        zero_accs = tuple(jnp.zeros((SQ_L, DH), jnp.float32) for _ in range(B * HQ))
        zero_ls = tuple(jnp.zeros((SQ_L, 1), jnp.float32) for _ in range(B * HQ))
        accs, ls = lax.fori_loop(0, N_DEV, chunk_step, (zero_accs, zero_ls))

        for b in range(B):
            ctx = jnp.concatenate(
                [accs[b * HQ + h] / ls[b * HQ + h] for h in range(HQ)], axis=1
            ).astype(jnp.bfloat16)
            out_ref[b] = jnp.dot(ctx, wo_b, preferred_element_type=jnp.float32)

    return pl.pallas_call(
        body,
        out_shape=jax.ShapeDtypeStruct((B, SQ_L, HQ * DH * 2), jnp.float32),
        in_specs=[
            pl.BlockSpec(memory_space=pltpu.VMEM),
            pl.BlockSpec(memory_space=pltpu.VMEM),
            pl.BlockSpec(memory_space=pltpu.VMEM),
            pl.BlockSpec(memory_space=pltpu.VMEM),
        ],
        out_specs=pl.BlockSpec(memory_space=pltpu.VMEM),
        scratch_shapes=[
            pltpu.VMEM((N_DEV, 2, B, HQ, SKV_L, DH), jnp.bfloat16),
            pltpu.SemaphoreType.DMA((PLANE - 1,)),
            pltpu.SemaphoreType.DMA((PLANE - 1,)),
            pltpu.SemaphoreType.DMA((PLANE,)),
            pltpu.SemaphoreType.DMA((PLANE,)),
            pltpu.SemaphoreType.DMA((PLANE,)),
            pltpu.SemaphoreType.DMA((PLANE,)),
            pltpu.SemaphoreType.DMA((2,)),
            pltpu.SemaphoreType.DMA((2,)),
            pltpu.SemaphoreType.DMA((2,)),
            pltpu.SemaphoreType.DMA((2,)),
        ],
        compiler_params=pltpu.CompilerParams(
            collective_id=0, vmem_limit_bytes=100 * 1024 * 1024
        ),
    )(x, Wq, kvt, Wo)


# device time: 337271 ns/iter; 1.2550x vs baseline; 1.0620x over previous
import jax
import jax.numpy as jnp
from jax import lax
from jax.experimental import pallas as pl
from jax.experimental.pallas import tpu as pltpu

N_DEV = 32
PLANE = 8
N_PLANES = N_DEV // PLANE
SQ_L = 256
SKV_L = 256
HQ = 4
DH = 64
BLK = 64


def kernel(x, Wq, K_ext, V_ext, Wo):
    B = x.shape[0]

    kvt = jnp.stack([K_ext, V_ext]).astype(jnp.bfloat16).transpose(0, 1, 3, 2, 4)

    def body(x_ref, wq_ref, kvt_ref, wo_ref, out_ref, kvg_ref,
             p1s_sems, p1r_sems, u0s_sems, u0r_sems, d0s_sems, d0r_sems,
             us_sems, ur_sems, ds_sems, dr_sems):
        my = lax.axis_index("i")
        my_p = my % PLANE
        pbase = (my // PLANE) * PLANE
        my_z = my // PLANE
        zup = (my + PLANE) % N_DEV
        zdn = (my + N_DEV - PLANE) % N_DEV

        barrier_sem = pltpu.get_barrier_semaphore()
        mates = [pbase + (my_p + j) % PLANE for j in range(1, PLANE)]
        for nbr in mates + [zup, zdn]:
            pl.semaphore_signal(
                barrier_sem, inc=1,
                device_id=(nbr,), device_id_type=pl.DeviceIdType.MESH,
            )
        pl.semaphore_wait(barrier_sem, PLANE + 1)

        kvg_ref[my] = kvt_ref[...]

        zt = N_PLANES - 1
        up_ok = my_z < zt
        dn_ok = my_z > 0

        def _when(cond, *ops):
            @pl.when(cond)
            def _():
                for op in ops:
                    op()

        def _chunk_z(slot, origin):
            u = pltpu.make_async_remote_copy(
                src_ref=kvg_ref.at[origin], dst_ref=kvg_ref.at[origin],
                send_sem=u0s_sems.at[slot], recv_sem=u0r_sems.at[slot],
                device_id=(zup,), device_id_type=pl.DeviceIdType.MESH,
            )
            d = pltpu.make_async_remote_copy(
                src_ref=kvg_ref.at[origin], dst_ref=kvg_ref.at[origin],
                send_sem=d0s_sems.at[slot], recv_sem=d0r_sems.at[slot],
                device_id=(zdn,), device_id_type=pl.DeviceIdType.MESH,
            )
            return u, d

        zc = [_chunk_z(0, my)]
        _when(up_ok, zc[0][0].start)
        _when(dn_ok, zc[0][1].start)

        p1 = []
        for j in range(1, PLANE):
            d = pltpu.make_async_remote_copy(
                src_ref=kvg_ref.at[my],
                dst_ref=kvg_ref.at[my],
                send_sem=p1s_sems.at[j - 1],
                recv_sem=p1r_sems.at[j - 1],
                device_id=(mates[j - 1],),
                device_id_type=pl.DeviceIdType.MESH,
            )
            d.start()
            p1.append(d)
        for j in range(1, PLANE):
            p1[j - 1].wait_recv()
            mj = pbase + (my_p + PLANE - j) % PLANE
            zc.append(_chunk_z(j, mj))
            _when(up_ok, zc[j][0].start)
            _when(dn_ok, zc[j][1].start)
        for d in p1:
            d.wait_send()

        U, D = [None], [None]
        for r in (1, 2):
            s = r % 2
            bu = jnp.maximum(my_z - r, 0)
            U.append(pltpu.make_async_remote_copy(
                src_ref=kvg_ref.at[pl.ds(bu * PLANE, PLANE)],
                dst_ref=kvg_ref.at[pl.ds(bu * PLANE, PLANE)],
                send_sem=us_sems.at[s], recv_sem=ur_sems.at[s],
                device_id=(zup,), device_id_type=pl.DeviceIdType.MESH,
            ))
            bd = jnp.minimum(my_z + r, zt)
            D.append(pltpu.make_async_remote_copy(
                src_ref=kvg_ref.at[pl.ds(bd * PLANE, PLANE)],
                dst_ref=kvg_ref.at[pl.ds(bd * PLANE, PLANE)],
                send_sem=ds_sems.at[s], recv_sem=dr_sems.at[s],
                device_id=(zdn,), device_id_type=pl.DeviceIdType.MESH,
            ))
        sU = [up_ok, up_ok & (my_z >= 1), up_ok & (my_z >= 2)]
        sD = [dn_ok, dn_ok & (my_z + 1 <= zt), dn_ok & (my_z + 2 <= zt)]
        rU = [my_z >= r + 1 for r in range(3)]
        rD = [my_z + r + 1 <= zt for r in range(3)]

        base = my * SQ_L
        wq_b = wq_ref[...].astype(jnp.bfloat16)
        wo_b = wo_ref[...].astype(jnp.bfloat16)
        qb = (base + lax.broadcasted_iota(jnp.int32, (SQ_L, SKV_L), 0)) // BLK
        cb = lax.broadcasted_iota(jnp.int32, (SQ_L, SKV_L), 1) // BLK

        qs = []
        for b in range(B):
            q_all = jnp.dot(
                x_ref[b].astype(jnp.bfloat16), wq_b,
                preferred_element_type=jnp.float32,
            )
            for h in range(HQ):
                qs.append(q_all[:, h * DH:(h + 1) * DH].astype(jnp.bfloat16))

        def block_pass(pb, scale, carry):
            def chunk_step(i, c):
                accs, ls = c
                o = pb * PLANE + i
                kb = cb + (o * (SKV_L // BLK))
                mask = (qb == kb) | (kb == 0) | ((qb + kb) % 3 == 0)
                new_accs = []
                new_ls = []
                for b in range(B):
                    for h in range(HQ):
                        i2 = b * HQ + h
                        k = kvg_ref[o, 0, b, h]
                        v = kvg_ref[o, 1, b, h]
                        sc = lax.dot_general(
                            qs[i2], k, (((1,), (1,)), ((), ())),
                            preferred_element_type=jnp.float32,
                        ) * 0.125
                        w = jnp.where(mask, jnp.exp(sc), 0.0) * scale
                        new_accs.append(accs[i2] + lax.dot_general(
                            w.astype(jnp.bfloat16), v, (((1,), (0,)), ((), ())),
                            preferred_element_type=jnp.float32,
                        ))
                        new_ls.append(ls[i2] + jnp.sum(w, axis=1, keepdims=True))
                return tuple(new_accs), tuple(new_ls)
            return lax.fori_loop(0, PLANE, chunk_step, carry)

        carry = (
            tuple(jnp.zeros((SQ_L, DH), jnp.float32) for _ in range(B * HQ)),
            tuple(jnp.zeros((SQ_L, 1), jnp.float32) for _ in range(B * HQ)),
        )

        def flag(c):
            return jnp.where(c, 1.0, 0.0).astype(jnp.float32)

        for j in range(PLANE):
            _when(rU[0], zc[j][0].wait_recv)
        _when(sU[1], U[1].start)
        for j in range(PLANE):
            _when(rD[0], zc[j][1].wait_recv)
        _when(sD[1], D[1].start)
        carry = block_pass(my_z, 1.0, carry)
        carry = block_pass(jnp.maximum(my_z - 1, 0), flag(rU[0]), carry)
        carry = block_pass(jnp.minimum(my_z + 1, zt), flag(rD[0]), carry)
        _when(rU[1], U[1].wait_recv)
        _when(sU[2], U[2].start)
        _when(rD[1], D[1].wait_recv)
        _when(sD[2], D[2].start)
        carry = block_pass(jnp.maximum(my_z - 2, 0), flag(rU[1]), carry)
        carry = block_pass(jnp.minimum(my_z + 2, zt), flag(rD[1]), carry)
        _when(rU[2], U[2].wait_recv)
        _when(rD[2], D[2].wait_recv)
        carry = block_pass(jnp.maximum(my_z - 3, 0), flag(rU[2]), carry)
        carry = block_pass(jnp.minimum(my_z + 3, zt), flag(rD[2]), carry)
        accs, ls = carry

        for j in range(PLANE):
            _when(up_ok, zc[j][0].wait_send)
            _when(dn_ok, zc[j][1].wait_send)
        _when(sU[1], U[1].wait_send)
        _when(sD[1], D[1].wait_send)
        _when(sU[2], U[2].wait_send)
        _when(sD[2], D[2].wait_send)

        for b in range(B):
            ctx = jnp.concatenate(
                [accs[b * HQ + h] / ls[b * HQ + h] for h in range(HQ)], axis=1
            ).astype(jnp.bfloat16)
            out_ref[b] = jnp.dot(ctx, wo_b, preferred_element_type=jnp.float32)

    return pl.pallas_call(
        body,
        out_shape=jax.ShapeDtypeStruct((B, SQ_L, HQ * DH * 2), jnp.float32),
        in_specs=[
            pl.BlockSpec(memory_space=pltpu.VMEM),
            pl.BlockSpec(memory_space=pltpu.VMEM),
            pl.BlockSpec(memory_space=pltpu.VMEM),
            pl.BlockSpec(memory_space=pltpu.VMEM),
        ],
        out_specs=pl.BlockSpec(memory_space=pltpu.VMEM),
        scratch_shapes=[
            pltpu.VMEM((N_DEV, 2, B, HQ, SKV_L, DH), jnp.bfloat16),
            pltpu.SemaphoreType.DMA((PLANE - 1,)),
            pltpu.SemaphoreType.DMA((PLANE - 1,)),
            pltpu.SemaphoreType.DMA((PLANE,)),
            pltpu.SemaphoreType.DMA((PLANE,)),
            pltpu.SemaphoreType.DMA((PLANE,)),
            pltpu.SemaphoreType.DMA((PLANE,)),
            pltpu.SemaphoreType.DMA((2,)),
            pltpu.SemaphoreType.DMA((2,)),
            pltpu.SemaphoreType.DMA((2,)),
            pltpu.SemaphoreType.DMA((2,)),
        ],
        compiler_params=pltpu.CompilerParams(
            collective_id=0, vmem_limit_bytes=100 * 1024 * 1024
        ),
    )(x, Wq, kvt, Wo)
